# baseline (speedup 1.0000x reference)
import jax
import jax.numpy as jnp
from jax import lax
from jax.experimental import pallas as pl
from jax.experimental.pallas import tpu as pltpu
from jax.experimental.pallas import tpu_sc as plsc

NC = 2
NS = 16
NW = NC * NS
L = 16
C = 128
BI = 8
CW = 16


def _fill_vmem(ref, rows, cols, value):
    cpr = cols // L

    def body(i, _):
        r = i // cpr
        cc = i % cpr
        ref[r, pl.ds(cc * L, L)] = jnp.full((L,), value, jnp.float32)
        return 0

    lax.fori_loop(0, rows * cpr, body, 0, unroll=8)


def _seed_rows(zsrc, dst_ref, base, rpt):
    full, rem = divmod(rpt, C)
    for k in range(full):
        pltpu.sync_copy(zsrc, dst_ref.at[pl.ds(base + k * C, C)])
    if rem:
        pltpu.sync_copy(zsrc.at[pl.ds(0, rem)],
                        dst_ref.at[pl.ds(base + full * C, rem)])


def _make_sc_segsum(n_pad, chunks, d):
    rpt = n_pad // NS
    nb = chunks // BI
    assert nb * BI == chunks
    mesh = plsc.VectorSubcoreMesh(
        core_axis_name="c", subcore_axis_name="s", num_cores=NC, num_subcores=NS
    )
    scratch = [
        pltpu.VMEM_SHARED((n_pad, d), jnp.float32),
        pltpu.VMEM((2, BI, C), jnp.int32),
        pltpu.VMEM((2, BI, C), jnp.int32),
        pltpu.VMEM((2, C, d), jnp.float32),
        pltpu.SemaphoreType.DMA,
        pltpu.SemaphoreType.DMA,
        pltpu.SemaphoreType.DMA,
    ]

    def body(table_h, src_h, dst_h, part_h, acc, srcv, dstv, rows, gsem, isem,
             ssem):
        ci = lax.axis_index("c")
        si = lax.axis_index("s")
        wid = ci * NS + si
        base = si * rpt
        _fill_vmem(rows.at[0], C, d, 0.0)
        _seed_rows(rows.at[0], acc, base, rpt)
        plsc.subcore_barrier()

        def idx_load(blk, slot):
            pltpu.async_copy(src_h.at[wid, pl.ds(blk * BI, BI)],
                             srcv.at[slot], isem)
            pltpu.async_copy(dst_h.at[wid, pl.ds(blk * BI, BI)],
                             dstv.at[slot], isem)

        def idx_wait():
            pltpu.make_async_copy(src_h.at[wid, pl.ds(0, BI)],
                                  srcv.at[0], isem).wait()
            pltpu.make_async_copy(dst_h.at[wid, pl.ds(0, BI)],
                                  dstv.at[0], isem).wait()

        def g_start(islot, j, rslot):
            pltpu.async_copy(table_h.at[srcv.at[islot, j]], rows.at[rslot],
                             gsem)

        def g_wait(rslot):
            pltpu.make_async_copy(table_h.at[srcv.at[0, 0]], rows.at[rslot],
                                  gsem).wait()

        def s_start(rslot, islot, j):
            pltpu.async_copy(rows.at[rslot], acc.at[dstv.at[islot, j]], ssem,
                             add=True)

        def s_wait():
            pltpu.make_async_copy(rows.at[0], acc.at[dstv.at[0, 0]],
                                  ssem).wait()

        idx_load(0, 0)
        idx_wait()
        g_start(0, 0, 0)

        def block(blk, _):
            s = blk % 2

            @pl.when(blk + 1 < nb)
            def _():
                idx_load(blk + 1, 1 - s)

            for j in range(BI):
                cur = j % 2
                g_wait(cur)
                if j == 0:
                    @pl.when(blk > 0)
                    def _():
                        s_wait()
                else:
                    s_wait()
                if j + 1 < BI:
                    g_start(s, j + 1, 1 - cur)
                else:
                    @pl.when(blk + 1 < nb)
                    def _():
                        idx_wait()
                        g_start(1 - s, 0, 1 - cur)
                s_start(cur, s, j)
            return 0

        lax.fori_loop(0, nb, block, 0)
        s_wait()
        plsc.subcore_barrier()
        pltpu.sync_copy(acc.at[pl.ds(base, rpt)],
                        part_h.at[ci, pl.ds(base, rpt)])

    params = (pltpu.CompilerParams(use_tc_tiling_on_sc=False)
              if d < 128 else None)
    return pl.kernel(body, out_type=jax.ShapeDtypeStruct((NC, n_pad, d),
                                                         jnp.float32),
                     mesh=mesh, scratch_types=tuple(scratch),
                     compiler_params=params)


def _make_sc_count(n_pad, chunks):
    rpt = n_pad // NS
    nb = chunks // BI
    mesh = plsc.VectorSubcoreMesh(
        core_axis_name="c", subcore_axis_name="s", num_cores=NC, num_subcores=NS
    )
    scratch = [
        pltpu.VMEM_SHARED((n_pad, CW), jnp.float32),
        pltpu.VMEM((BI, C), jnp.int32),
        pltpu.VMEM((C, CW), jnp.float32),
        pltpu.VMEM((C, CW), jnp.float32),
    ]

    def body(dst_h, cnt_h, cacc, dstv, onesv, cstage):
        ci = lax.axis_index("c")
        si = lax.axis_index("s")
        wid = ci * NS + si
        base = si * rpt
        _fill_vmem(cstage, C, CW, 0.0)
        _seed_rows(cstage, cacc, base, rpt)
        _fill_vmem(onesv, C, CW, 1.0)
        plsc.subcore_barrier()

        def chunk(j, _):
            pltpu.sync_copy(onesv, cacc.at[dstv.at[j]], add=True)
            return 0

        for blk in range(nb):
            pltpu.sync_copy(dst_h.at[wid, pl.ds(blk * BI, BI)], dstv)
            lax.fori_loop(0, BI, chunk, 0)
        plsc.subcore_barrier()
        pltpu.sync_copy(cacc.at[pl.ds(base, rpt)],
                        cnt_h.at[ci, pl.ds(base, rpt)])

    return pl.kernel(body, out_type=jax.ShapeDtypeStruct((NC, n_pad, CW),
                                                         jnp.float32),
                     mesh=mesh, scratch_types=tuple(scratch),
                     compiler_params=pltpu.CompilerParams(
                         use_tc_tiling_on_sc=False))


def _dot(a, b):
    return jnp.dot(a, b, preferred_element_type=jnp.float32,
                   precision=lax.Precision.DEFAULT)


RB = 2000


def _full(shape):
    return pl.BlockSpec(shape, lambda p, i: (0,) * len(shape))


def _linear(hprev, w, b, n):
    h = w.shape[1]
    dh = hprev.shape[1]
    nb = n // RB

    def body(hp_r, w_r, b_r, u_r):
        u_r[...] = _dot(hp_r[...], w_r[...]) + b_r[...]

    return pl.pallas_call(
        body,
        grid=(nb,),
        in_specs=[pl.BlockSpec((RB, dh), lambda i: (i, 0)),
                  pl.BlockSpec(w.shape, lambda i: (0, 0)),
                  pl.BlockSpec(b.shape, lambda i: (0, 0))],
        out_specs=pl.BlockSpec((RB, h), lambda i: (i, 0)),
        out_shape=jax.ShapeDtypeStruct((n, h), jnp.float32),
    )(hprev, w, b)


def _combine(part, cpart, u, w_mean, g, be, wnext, bfin, n, final):
    h = u.shape[1]
    d = part.shape[2]
    hn = wnext.shape[1]
    nb = n // RB
    inv_n = 1.0 / n

    def p0_map3(p, i):
        return (0, jnp.where(p == 0, i, 0), 0)

    def p0_map2(p, i):
        return (jnp.where(p == 0, i, 0), 0)

    def body(part_r, cpart_r, u_r, *rest):
        if w_mean is not None:
            wm_r = rest[0]
            rest = rest[1:]
        (g_r, be_r, wn_r) = rest[:3]
        rest = rest[3:]
        if final:
            (bf_r, out_r, t_s, st_s) = rest
        else:
            (h_r, y_r, t_s, st_s) = rest
        p = pl.program_id(0)
        i = pl.program_id(1)

        @pl.when(p == 0)
        def _():
            cp = cpart_r[...]
            rec = 1.0 / jnp.maximum(cp[0] + cp[1], 1.0)
            mean = (part_r[0] + part_r[1]) * rec[:, 0:1]
            if w_mean is not None:
                t = _dot(mean, wm_r[...]) + u_r[...]
            else:
                t = mean + u_r[...]
            t_s[pl.ds(i * RB, RB), :] = t

            @pl.when(i == 0)
            def _():
                st_s[...] = jnp.zeros_like(st_s)

            st_s[0:1, :] += jnp.sum(t, axis=0, keepdims=True)
            st_s[1:2, :] += jnp.sum(t * t, axis=0, keepdims=True)

        @pl.when(p == 1)
        def _():
            t = t_s[pl.ds(i * RB, RB), :]
            mu = st_s[0:1, :] * inv_n
            var = st_s[1:2, :] * inv_n - mu * mu
            hh = jnp.maximum(
                g_r[...] * ((t - mu) * lax.rsqrt(var + 1e-5)) + be_r[...],
                0.0)
            if final:
                out_r[...] = jnp.maximum(_dot(hh, wn_r[...]) + bf_r[...], 0.0)
            else:
                h_r[...] = hh
                y_r[...] = _dot(hh, wn_r[...])

    in_specs = [
        pl.BlockSpec((2, RB, d), p0_map3),
        pl.BlockSpec((2, RB, CW), p0_map3),
        pl.BlockSpec((RB, h), p0_map2),
    ]
    args = [part, cpart, u]
    if w_mean is not None:
        in_specs.append(_full(w_mean.shape))
        args.append(w_mean)
    in_specs += [_full((1, h)), _full((1, h)), _full(wnext.shape)]
    args += [g, be, wnext]
    def p1_map(p, i):
        return (jnp.where(p == 0, 0, i), 0)

    if final:
        in_specs.append(_full(bfin.shape))
        args.append(bfin)
        out_specs = pl.BlockSpec((RB, hn), p1_map)
        out_shape = jax.ShapeDtypeStruct((n, hn), jnp.float32)
    else:
        out_specs = (pl.BlockSpec((RB, h), p1_map),
                     pl.BlockSpec((RB, hn), p1_map))
        out_shape = (jax.ShapeDtypeStruct((n, h), jnp.float32),
                     jax.ShapeDtypeStruct((n, hn), jnp.float32))
    return pl.pallas_call(
        body,
        grid=(2, nb),
        in_specs=in_specs,
        out_specs=out_specs,
        out_shape=out_shape,
        scratch_shapes=[pltpu.VMEM((n, h), jnp.float32),
                        pltpu.VMEM((8, h), jnp.float32)],
    )(*args)


def kernel(x, edge_index, Wl1, Wr1, b1, g1, be1, Wl2, Wr2, b2, g2, be2,
           Wl3, Wr3, b3, g3, be3, Wl4, Wr4, b4, g4, be4, Wf, bf):
    n = x.shape[0]
    e = edge_index.shape[1]

    n_pad = -(-n // (NS * 8)) * (NS * 8)
    chunks = -(-e // (NW * BI * C)) * BI
    e_pad = NW * chunks * C
    src = edge_index[0].astype(jnp.int32)
    dst = edge_index[1].astype(jnp.int32)
    pad = e_pad - e
    if pad:
        pr = max(n_pad - n, 1)
        ar = jnp.arange(pad, dtype=jnp.int32)
        src = jnp.concatenate([src, ar % n])
        dst = jnp.concatenate([dst, n + ar % pr])
    src3 = src.reshape(NW, chunks, C)
    dst3 = dst.reshape(NW, chunks, C)

    cpart = _make_sc_count(n_pad, chunks)(dst3)
    part1 = _make_sc_segsum(n_pad, chunks, x.shape[1])(x, src3, dst3)
    u1 = _linear(x, Wr1, b1.reshape(1, -1), n)
    h1, y2 = _combine(part1, cpart, u1, Wl1, g1.reshape(1, -1),
                      be1.reshape(1, -1), Wl2, None, n, final=False)

    part2 = _make_sc_segsum(n_pad, chunks, y2.shape[1])(y2, src3, dst3)
    u2 = _linear(h1, Wr2, b2.reshape(1, -1), n)
    h2, y3 = _combine(part2, cpart, u2, None, g2.reshape(1, -1),
                      be2.reshape(1, -1), Wl3, None, n, final=False)

    part3 = _make_sc_segsum(n_pad, chunks, y3.shape[1])(y3, src3, dst3)
    u3 = _linear(h2, Wr3, b3.reshape(1, -1), n)
    h3, y4 = _combine(part3, cpart, u3, None, g3.reshape(1, -1),
                      be3.reshape(1, -1), Wl4, None, n, final=False)

    part4 = _make_sc_segsum(n_pad, chunks, y4.shape[1])(y4, src3, dst3)
    u4 = _linear(h3, Wr4, b4.reshape(1, -1), n)
    return _combine(part4, cpart, u4, None, g4.reshape(1, -1),
                    be4.reshape(1, -1), Wf, bf.reshape(1, -1), n, final=True)
